# baseline (speedup 1.0000x reference)
import functools

import jax
import jax.numpy as jnp
from jax import lax
from jax.experimental import pallas as pl
from jax.experimental.pallas import tpu as pltpu
from jax.experimental.pallas import tpu_sc as plsc

_VOCAB = 1000000
_D = 64
_B = 4096
_T = 200
_N = _B * _T

_NC = 2
_NS = 16
_NW = _NC * _NS
_BB = _B // _NW

_RB = 4096
_HALF = _RB // 2
_TGRID = -(-_VOCAB // _RB)
_VP = _TGRID * _RB

_mesh = plsc.VectorSubcoreMesh(core_axis_name="c", subcore_axis_name="s")


def _transpose_body(t_ref, o_ref):
    x = t_ref[...]
    xt = jnp.transpose(x)
    lo = lax.slice(xt, (0, 0), (_HALF, _D))
    hi = lax.slice(xt, (_HALF, 0), (_RB, _D))
    o_ref[...] = jnp.concatenate([lo, hi], axis=1)


_transpose_tc = pl.pallas_call(
    _transpose_body,
    grid=(_TGRID,),
    in_specs=[pl.BlockSpec((_D, _RB), lambda i: (0, i))],
    out_specs=pl.BlockSpec((_HALF, 128), lambda i: (i, 0)),
    out_shape=jax.ShapeDtypeStruct((_VP // 2, 128), jnp.float32),
)


@functools.partial(
    pl.kernel,
    mesh=_mesh,
    out_type=jax.ShapeDtypeStruct((_T, 8, _NW, 1024), jnp.float32),
    compiler_params=pltpu.CompilerParams(
        use_tc_tiling_on_sc=False, needs_layout_passes=False),
    scratch_types=[
        pltpu.VMEM((_T, _BB), jnp.int32),
        pltpu.VMEM((_BB,), jnp.int32),
        pltpu.VMEM((_BB,), jnp.int32),
        pltpu.VMEM((_BB, _D), jnp.float32),
        pltpu.VMEM((_BB, _D), jnp.float32),
        pltpu.VMEM((_BB * _D,), jnp.float32),
        pltpu.VMEM((_BB * _D,), jnp.float32),
        pltpu.VMEM((1088,), jnp.float32),
        pltpu.VMEM((1088,), jnp.float32),
        pltpu.SemaphoreType.DMA,
        pltpu.SemaphoreType.DMA,
        pltpu.SemaphoreType.DMA,
        pltpu.SemaphoreType.DMA,
    ],
)
def _gather_kernel(idx_hbm, table_hbm, out_hbm, idx_v, k0, k1, rb0, rb1,
                   tb0, tb1, s0, s1, g0, g1, w0, w1):
    wid = lax.axis_index("s") * _NC + lax.axis_index("c")

    pltpu.sync_copy(idx_hbm.at[:, pl.ds(wid * _BB, _BB)], idx_v)

    iota = lax.iota(jnp.int32, 16)
    skew = [iota * 17 + j for j in range(16)]

    def remap(t, kbuf):
        for g in range(_BB // 16):
            v = idx_v[t, pl.ds(g * 16, 16)]
            blk = lax.shift_left(lax.shift_right_logical(v, 12), 12)
            k = (blk + lax.shift_left(v & 2047, 1)
                 + (lax.shift_right_logical(v, 11) & 1))
            kbuf[pl.ds(g * 16, 16)] = k

    def fire_gather(kbuf, rbuf, gsem):
        pltpu.make_async_copy(table_hbm.at[kbuf], rbuf, gsem).start()

    def wait_gather(kbuf, rbuf, gsem):
        pltpu.make_async_copy(table_hbm.at[kbuf], rbuf, gsem).wait()

    def transpose(rbuf, tbuf, s0, s1):
        def lbody(i, carry):
            l0 = i * 16
            sbuf = s0
            for l in range(16):
                for ci in range(4):
                    sbuf[pl.ds(ci * 272 + l * 17, 16)] = (
                        rbuf[l0 + l, pl.ds(ci * 16, 16)])
            for ci, c0 in enumerate(range(0, _D, 16)):
                for j in range(16):
                    c = c0 + j
                    col = plsc.load_gather(sbuf, [skew[j] + ci * 272])
                    tbuf[pl.ds((c // 8) * 1024 + (c % 8) * 128 + l0, 16)] = col
            return carry

        lax.fori_loop(0, _BB // 16, lbody, 0)

    def fire_writes(t, tbuf, wsem):
        for a in range(8):
            pltpu.make_async_copy(
                tbuf.at[pl.ds(a * 1024, 1024)], out_hbm.at[t, a, wid], wsem
            ).start()

    def wait_writes(t, tbuf, wsem):
        for a in range(8):
            pltpu.make_async_copy(
                tbuf.at[pl.ds(a * 1024, 1024)], out_hbm.at[t, a, wid], wsem
            ).wait()

    remap(0, k0)
    fire_gather(k0, rb0, g0)
    remap(1, k1)
    fire_gather(k1, rb1, g1)

    def body(t2, carry):
        for b, (kb, rbuf, tbuf, gsem, wsem) in enumerate(
                ((k0, rb0, tb0, g0, w0), (k1, rb1, tb1, g1, w1))):
            t = t2 * 2 + b
            wait_gather(kb, rbuf, gsem)

            @pl.when(t >= 2)
            def _():
                wait_writes(t - 2, tbuf, wsem)

            transpose(rbuf, tbuf, s0, s1)
            fire_writes(t, tbuf, wsem)

            @pl.when(t + 2 < _T)
            def _():
                remap(t + 2, kb)
                fire_gather(kb, rbuf, gsem)

        return carry

    lax.fori_loop(0, _T // 2, body, 0)

    wait_writes(_T - 2, tb0, w0)
    wait_writes(_T - 1, tb1, w1)


def kernel(idx, table):
    table_cm = jnp.transpose(table)
    packed = _transpose_tc(table_cm)
    table_lin = jnp.reshape(packed, (_VP, _D))
    idx_tm = jnp.transpose(jnp.asarray(idx, jnp.int32))
    out5 = _gather_kernel(idx_tm, table_lin)
    out5 = out5.reshape(_T, 8, _NW, 8, 128)
    return jnp.transpose(out5, (2, 4, 0, 1, 3)).reshape(_B, _T, _D)


# speedup vs baseline: 2.1605x; 2.1605x over previous
import functools

import jax
import jax.numpy as jnp
from jax import lax
from jax.experimental import pallas as pl
from jax.experimental.pallas import tpu as pltpu
from jax.experimental.pallas import tpu_sc as plsc

_VOCAB = 1000000
_D = 64
_B = 4096
_T = 200
_N = _B * _T

_NC = 2
_NS = 16
_NW = _NC * _NS
_BB = _B // _NW

_RB = 4096
_HALF = _RB // 2
_TGRID = -(-_VOCAB // _RB)
_VP = _TGRID * _RB

_mesh = plsc.VectorSubcoreMesh(core_axis_name="c", subcore_axis_name="s")


def _transpose_body(t_ref, o_ref):
    x = t_ref[...]
    xt = jnp.transpose(x)
    lo = lax.slice(xt, (0, 0), (_HALF, _D))
    hi = lax.slice(xt, (_HALF, 0), (_RB, _D))
    o_ref[...] = jnp.concatenate([lo, hi], axis=1)


_transpose_tc = pl.pallas_call(
    _transpose_body,
    grid=(_TGRID,),
    in_specs=[pl.BlockSpec((_D, _RB), lambda i: (0, i))],
    out_specs=pl.BlockSpec((_HALF, 128), lambda i: (i, 0)),
    out_shape=jax.ShapeDtypeStruct((_VP // 2, 128), jnp.float32),
)


@functools.partial(
    pl.kernel,
    mesh=_mesh,
    out_type=jax.ShapeDtypeStruct((_T, 8, _NW, 1024), jnp.float32),
    compiler_params=pltpu.CompilerParams(
        use_tc_tiling_on_sc=False, needs_layout_passes=False),
    scratch_types=[
        pltpu.VMEM((_T, _BB), jnp.int32),
        pltpu.VMEM((_BB,), jnp.int32),
        pltpu.VMEM((_BB,), jnp.int32),
        pltpu.VMEM((_BB, _D), jnp.float32),
        pltpu.VMEM((_BB, _D), jnp.float32),
        pltpu.VMEM((_BB * _D,), jnp.float32),
        pltpu.VMEM((_BB * _D,), jnp.float32),
        pltpu.VMEM((8 * 1088,), jnp.float32),
        pltpu.VMEM((16,), jnp.float32),
        pltpu.SemaphoreType.DMA,
        pltpu.SemaphoreType.DMA,
        pltpu.SemaphoreType.DMA,
        pltpu.SemaphoreType.DMA,
    ],
)
def _gather_kernel(idx_hbm, table_hbm, out_hbm, idx_v, k0, k1, rb0, rb1,
                   tb0, tb1, s0, s1, g0, g1, w0, w1):
    wid = lax.axis_index("s") * _NC + lax.axis_index("c")

    pltpu.sync_copy(idx_hbm.at[:, pl.ds(wid * _BB, _BB)], idx_v)

    iota = lax.iota(jnp.int32, 16)
    skew = [iota * 17 + j for j in range(16)]

    def remap(t, kbuf):
        for g in range(_BB // 16):
            v = idx_v[t, pl.ds(g * 16, 16)]
            blk = lax.shift_left(lax.shift_right_logical(v, 12), 12)
            k = (blk + lax.shift_left(v & 2047, 1)
                 + (lax.shift_right_logical(v, 11) & 1))
            kbuf[pl.ds(g * 16, 16)] = k

    def fire_gather(kbuf, rbuf, gsem):
        pltpu.make_async_copy(table_hbm.at[kbuf], rbuf, gsem).start()

    def wait_gather(kbuf, rbuf, gsem):
        pltpu.make_async_copy(table_hbm.at[kbuf], rbuf, gsem).wait()

    def transpose(rbuf, tbuf, s0, s1):
        @functools.partial(plsc.parallel_loop, 0, _BB // 16)
        def lbody(i):
            l0 = i * 16
            seg = i * 1088
            sbuf = s0
            for l in range(16):
                for ci in range(4):
                    sbuf[pl.ds(seg + ci * 272 + l * 17, 16)] = (
                        rbuf[l0 + l, pl.ds(ci * 16, 16)])
            for ci, c0 in enumerate(range(0, _D, 16)):
                for j in range(16):
                    c = c0 + j
                    col = plsc.load_gather(sbuf, [skew[j] + (seg + ci * 272)])
                    tbuf[pl.ds((c // 8) * 1024 + (c % 8) * 128 + l0, 16)] = col

    def fire_writes(t, tbuf, wsem):
        for a in range(8):
            pltpu.make_async_copy(
                tbuf.at[pl.ds(a * 1024, 1024)], out_hbm.at[t, a, wid], wsem
            ).start()

    def wait_writes(t, tbuf, wsem):
        for a in range(8):
            pltpu.make_async_copy(
                tbuf.at[pl.ds(a * 1024, 1024)], out_hbm.at[t, a, wid], wsem
            ).wait()

    remap(0, k0)
    fire_gather(k0, rb0, g0)
    remap(1, k1)
    fire_gather(k1, rb1, g1)

    def body(t2, carry):
        for b, (kb, rbuf, tbuf, gsem, wsem) in enumerate(
                ((k0, rb0, tb0, g0, w0), (k1, rb1, tb1, g1, w1))):
            t = t2 * 2 + b
            wait_gather(kb, rbuf, gsem)

            @pl.when(t >= 2)
            def _():
                wait_writes(t - 2, tbuf, wsem)

            transpose(rbuf, tbuf, s0, s1)
            fire_writes(t, tbuf, wsem)

            @pl.when(t + 2 < _T)
            def _():
                remap(t + 2, kb)
                fire_gather(kb, rbuf, gsem)

        return carry

    lax.fori_loop(0, _T // 2, body, 0)

    wait_writes(_T - 2, tb0, w0)
    wait_writes(_T - 1, tb1, w1)


def kernel(idx, table):
    table_cm = jnp.transpose(table)
    packed = _transpose_tc(table_cm)
    table_lin = jnp.reshape(packed, (_VP, _D))
    idx_tm = jnp.transpose(jnp.asarray(idx, jnp.int32))
    out5 = _gather_kernel(idx_tm, table_lin)
    out5 = out5.reshape(_T, 8, _NW, 8, 128)
    return jnp.transpose(out5, (2, 4, 0, 1, 3)).reshape(_B, _T, _D)
